# baseline (speedup 1.0000x reference)
import jax
import jax.numpy as jnp
from jax.experimental import pallas as pl
from jax.experimental.pallas import tpu as pltpu

_CLS_PAD = 128


def _heads_kernel(x_ref, w_ref, bc_ref, bb_ref, sc_ref, pd_ref):
    kc = sc_ref.shape[1]
    y = jnp.dot(x_ref[...], w_ref[...], preferred_element_type=jnp.float32)
    sc_ref[...] = y[:, :kc] + bc_ref[...]
    pd_ref[...] = y[:, _CLS_PAD:] + bb_ref[...]


def kernel(x, W_cls, b_cls, W_box, b_box):
    if x.ndim > 2:
        x = x.reshape(x.shape[0], -1)
    n, d = x.shape
    kc = W_cls.shape[1]
    kb = W_box.shape[1]
    bm = 1000
    assert n % bm == 0 and kc <= _CLS_PAD

    w_cat = jnp.concatenate(
        [jnp.pad(W_cls, ((0, 0), (0, _CLS_PAD - kc))), W_box], axis=1)
    bc2 = b_cls.reshape(1, kc)
    bb2 = b_box.reshape(1, kb)

    scores, deltas = pl.pallas_call(
        _heads_kernel,
        grid=(n // bm,),
        in_specs=[
            pl.BlockSpec((bm, d), lambda i: (i, 0)),
            pl.BlockSpec((d, _CLS_PAD + kb), lambda i: (0, 0)),
            pl.BlockSpec((1, kc), lambda i: (0, 0)),
            pl.BlockSpec((1, kb), lambda i: (0, 0)),
        ],
        out_specs=[
            pl.BlockSpec((bm, kc), lambda i: (i, 0)),
            pl.BlockSpec((bm, kb), lambda i: (i, 0)),
        ],
        out_shape=[
            jax.ShapeDtypeStruct((n, kc), jnp.float32),
            jax.ShapeDtypeStruct((n, kb), jnp.float32),
        ],
        compiler_params=pltpu.CompilerParams(
            dimension_semantics=("parallel",),
        ),
    )(x, w_cat, bc2, bb2)
    return (scores, deltas)


# speedup vs baseline: 1.0188x; 1.0188x over previous
import jax
import jax.numpy as jnp
from jax.experimental import pallas as pl
from jax.experimental.pallas import tpu as pltpu

_CLS_PAD = 128
_C = 5
_BM = 2000


def _heads_kernel(*refs):
    x_refs = refs[:_C]
    w_ref, bc_ref, bb_ref, sc_ref, pd_ref = refs[_C:]
    kc = sc_ref.shape[1]
    bs = x_refs[0].shape[0]
    w = w_ref[...]
    for r, x_ref in enumerate(x_refs):
        y = jnp.dot(x_ref[...], w, preferred_element_type=jnp.float32)
        sc_ref[pl.ds(r * bs, bs), :] = y[:, :kc] + bc_ref[...]
        pd_ref[pl.ds(r * bs, bs), :] = y[:, _CLS_PAD:] + bb_ref[...]


def kernel(x, W_cls, b_cls, W_box, b_box):
    if x.ndim > 2:
        x = x.reshape(x.shape[0], -1)
    n, d = x.shape
    kc = W_cls.shape[1]
    kb = W_box.shape[1]
    bs = _BM // _C
    assert n % _BM == 0 and kc <= _CLS_PAD

    w_cat = jnp.concatenate(
        [jnp.pad(W_cls, ((0, 0), (0, _CLS_PAD - kc))), W_box], axis=1)
    bc2 = b_cls.reshape(1, kc)
    bb2 = b_box.reshape(1, kb)

    x_specs = [
        pl.BlockSpec((bs, d), lambda i, r=r: (_C * i + r, 0))
        for r in range(_C)
    ]
    scores, deltas = pl.pallas_call(
        _heads_kernel,
        grid=(n // _BM,),
        in_specs=x_specs + [
            pl.BlockSpec((d, _CLS_PAD + kb), lambda i: (0, 0)),
            pl.BlockSpec((1, kc), lambda i: (0, 0)),
            pl.BlockSpec((1, kb), lambda i: (0, 0)),
        ],
        out_specs=[
            pl.BlockSpec((_BM, kc), lambda i: (i, 0)),
            pl.BlockSpec((_BM, kb), lambda i: (i, 0)),
        ],
        out_shape=[
            jax.ShapeDtypeStruct((n, kc), jnp.float32),
            jax.ShapeDtypeStruct((n, kb), jnp.float32),
        ],
        compiler_params=pltpu.CompilerParams(
            dimension_semantics=("parallel",),
        ),
    )(*([x] * _C), w_cat, bc2, bb2)
    return (scores, deltas)


# speedup vs baseline: 1.0580x; 1.0385x over previous
import jax
import jax.numpy as jnp
from jax.experimental import pallas as pl
from jax.experimental.pallas import tpu as pltpu

_CLS_PAD = 128
_BS = 400
_NBUF = 10


def _mm_kernel(x_hbm, w_ref, bc_ref, bb_ref, sc_hbm, pd_hbm,
               x_buf, sc_buf, pd_buf, x_sem, sc_sem, pd_sem):
    nchunks = x_hbm.shape[0] // _BS
    kc = sc_hbm.shape[1]
    i = pl.program_id(0)
    s = jax.lax.rem(i, _NBUF)

    def x_copy(c, slot):
        return pltpu.make_async_copy(
            x_hbm.at[pl.ds(c * _BS, _BS), :], x_buf.at[slot], x_sem.at[slot])

    def sc_copy(c, slot):
        return pltpu.make_async_copy(
            sc_buf.at[slot], sc_hbm.at[pl.ds(c * _BS, _BS), :], sc_sem.at[slot])

    def pd_copy(c, slot):
        return pltpu.make_async_copy(
            pd_buf.at[slot], pd_hbm.at[pl.ds(c * _BS, _BS), :], pd_sem.at[slot])

    @pl.when(i == 0)
    def _prologue():
        for k in range(_NBUF):
            x_copy(k, k).start()

    x_copy(i, s).wait()

    @pl.when(i >= _NBUF)
    def _drain():
        sc_copy(i - _NBUF, s).wait()
        pd_copy(i - _NBUF, s).wait()

    y = jnp.dot(x_buf[s].astype(jnp.bfloat16), w_ref[...],
                preferred_element_type=jnp.float32)
    sc_buf[s] = y[:, :kc] + bc_ref[...]
    pd_buf[s] = y[:, _CLS_PAD:] + bb_ref[...]
    sc_copy(i, s).start()
    pd_copy(i, s).start()

    @pl.when(i + _NBUF < nchunks)
    def _prefetch():
        x_copy(i + _NBUF, s).start()

    @pl.when(i == nchunks - 1)
    def _epilogue():
        for k in range(_NBUF):
            c = nchunks - _NBUF + k
            slot = jax.lax.rem(jnp.int32(c), _NBUF)
            sc_copy(c, slot).wait()
            pd_copy(c, slot).wait()


def kernel(x, W_cls, b_cls, W_box, b_box):
    if x.ndim > 2:
        x = x.reshape(x.shape[0], -1)
    n, d = x.shape
    kc = W_cls.shape[1]
    kb = W_box.shape[1]
    assert n % _BS == 0 and n // _BS >= _NBUF and kc <= _CLS_PAD

    w_cat = jnp.concatenate(
        [jnp.pad(W_cls, ((0, 0), (0, _CLS_PAD - kc))), W_box],
        axis=1).astype(jnp.bfloat16)
    bc2 = b_cls.reshape(1, kc)
    bb2 = b_box.reshape(1, kb)

    scores, deltas = pl.pallas_call(
        _mm_kernel,
        grid=(n // _BS,),
        in_specs=[
            pl.BlockSpec(memory_space=pl.ANY),
            pl.BlockSpec((d, _CLS_PAD + kb), lambda i: (0, 0)),
            pl.BlockSpec((1, kc), lambda i: (0, 0)),
            pl.BlockSpec((1, kb), lambda i: (0, 0)),
        ],
        out_specs=[
            pl.BlockSpec(memory_space=pl.ANY),
            pl.BlockSpec(memory_space=pl.ANY),
        ],
        out_shape=[
            jax.ShapeDtypeStruct((n, kc), jnp.float32),
            jax.ShapeDtypeStruct((n, kb), jnp.float32),
        ],
        scratch_shapes=[
            pltpu.VMEM((_NBUF, _BS, d), jnp.float32),
            pltpu.VMEM((_NBUF, _BS, kc), jnp.float32),
            pltpu.VMEM((_NBUF, _BS, kb), jnp.float32),
            pltpu.SemaphoreType.DMA((_NBUF,)),
            pltpu.SemaphoreType.DMA((_NBUF,)),
            pltpu.SemaphoreType.DMA((_NBUF,)),
        ],
        compiler_params=pltpu.CompilerParams(
            dimension_semantics=("arbitrary",),
        ),
    )(x, w_cat, bc2, bb2)
    return (scores, deltas)
